# baseline (speedup 1.0000x reference)
import functools

import jax
import jax.numpy as jnp
from jax import lax
from jax.experimental import pallas as pl
from jax.experimental.pallas import tpu as pltpu
from jax.experimental.pallas import tpu_sc as plsc

NUM_CORES = 2
NUM_SUBCORES = 16
NUM_WORKERS = NUM_CORES * NUM_SUBCORES

CHUNK = 512


def _make_gather(total_rows: int, dim: int):
  assert total_rows % (NUM_WORKERS * CHUNK) == 0
  per_w = total_rows // NUM_WORKERS
  n_chunks = per_w // CHUNK
  mesh = plsc.VectorSubcoreMesh(core_axis_name="c", subcore_axis_name="s")

  @functools.partial(
      pl.kernel,
      out_type=jax.ShapeDtypeStruct((total_rows, dim), jnp.float32),
      mesh=mesh,
      scratch_types=[
          pltpu.VMEM((CHUNK,), jnp.int32),
          pltpu.VMEM((CHUNK, dim), jnp.float32),
          pltpu.SemaphoreType.DMA,
      ],
      compiler_params=pltpu.CompilerParams(use_tc_tiling_on_sc=False),
  )
  def gather_kernel(idx_hbm, table_hbm, out_hbm, idx_v, rows_v, sem):
    wid = lax.axis_index("s") * NUM_CORES + lax.axis_index("c")
    base = wid * per_w

    def body(g, _):
      off = base + g * CHUNK
      pltpu.sync_copy(idx_hbm.at[pl.ds(off, CHUNK)], idx_v)
      pltpu.async_copy(table_hbm.at[idx_v], rows_v, sem).wait()
      pltpu.sync_copy(rows_v, out_hbm.at[pl.ds(off, CHUNK)])
      return ()

    lax.fori_loop(0, n_chunks, body, (), unroll=False)

  return gather_kernel


def kernel(input_ids, table):
  b, s = input_ids.shape
  flat = input_ids.reshape(b * s).astype(jnp.int32)
  out = _make_gather(b * s, table.shape[1])(flat, table)
  return out.reshape(b, s, table.shape[1])


# speedup vs baseline: 1.0389x; 1.0389x over previous
import functools

import jax
import jax.numpy as jnp
from jax import lax
from jax.experimental import pallas as pl
from jax.experimental.pallas import tpu as pltpu
from jax.experimental.pallas import tpu_sc as plsc

NUM_CORES = 2
NUM_SUBCORES = 16
NUM_WORKERS = NUM_CORES * NUM_SUBCORES

CHUNK = 320
NBUF = 4


def _make_gather(total_rows: int, dim: int):
  per_w = total_rows // NUM_WORKERS
  assert per_w * NUM_WORKERS == total_rows
  n_chunks = per_w // CHUNK
  assert n_chunks * CHUNK == per_w
  assert n_chunks > NBUF and (n_chunks - NBUF) % NBUF == 0
  mesh = plsc.VectorSubcoreMesh(core_axis_name="c", subcore_axis_name="s")

  @functools.partial(
      pl.kernel,
      out_type=jax.ShapeDtypeStruct((total_rows, dim), jnp.float32),
      mesh=mesh,
      scratch_types=[
          pltpu.VMEM((per_w,), jnp.int32),
          [pltpu.VMEM((CHUNK, dim), jnp.float32) for _ in range(NBUF)],
          [pltpu.SemaphoreType.DMA for _ in range(NBUF)],
          [pltpu.SemaphoreType.DMA for _ in range(NBUF)],
      ],
      compiler_params=pltpu.CompilerParams(use_tc_tiling_on_sc=False),
  )
  def gather_kernel(idx_hbm, table_hbm, out_hbm, idx_v, rows, gsem, ssem):
    wid = lax.axis_index("s") * NUM_CORES + lax.axis_index("c")
    base = wid * per_w

    pltpu.sync_copy(idx_hbm.at[pl.ds(base, per_w)], idx_v)

    def fire_gather(c, b):
      pltpu.async_copy(
          table_hbm.at[idx_v.at[pl.ds(c * CHUNK, CHUNK)]], rows[b], gsem[b])

    def fire_store(c, b):
      pltpu.async_copy(rows[b], out_hbm.at[pl.ds(base + c * CHUNK, CHUNK)],
                       ssem[b])

    def wait_gather(b):
      pltpu.make_async_copy(
          table_hbm.at[pl.ds(0, CHUNK)], rows[b], gsem[b]).wait()

    def wait_store(b):
      pltpu.make_async_copy(
          rows[b], out_hbm.at[pl.ds(0, CHUNK)], ssem[b]).wait()

    for b in range(NBUF):
      fire_gather(b, b)

    @pl.loop(0, n_chunks - NBUF, step=NBUF)
    def _(g0):
      for b in range(NBUF):
        wait_gather(b)
        fire_store(g0 + b, b)
      for b in range(NBUF):
        wait_store(b)
        fire_gather(g0 + b + NBUF, b)

    for b in range(NBUF):
      c = n_chunks - NBUF + b
      wait_gather(b)
      fire_store(c, b)
    for b in range(NBUF):
      wait_store(b)

  return gather_kernel


def kernel(input_ids, table):
  b, s = input_ids.shape
  flat = input_ids.reshape(b * s).astype(jnp.int32)
  out = _make_gather(b * s, table.shape[1])(flat, table)
  return out.reshape(b, s, table.shape[1])


# speedup vs baseline: 1.0662x; 1.0263x over previous
import functools

import jax
import jax.numpy as jnp
from jax import lax
from jax.experimental import pallas as pl
from jax.experimental.pallas import tpu as pltpu
from jax.experimental.pallas import tpu_sc as plsc

NUM_CORES = 2
NUM_SUBCORES = 16
NUM_WORKERS = NUM_CORES * NUM_SUBCORES

CHUNK = 320
NBUF = 4


def _make_gather(total_rows: int, dim: int):
  per_w = total_rows // NUM_WORKERS
  assert per_w * NUM_WORKERS == total_rows
  n_chunks = per_w // CHUNK
  assert n_chunks * CHUNK == per_w
  assert n_chunks > NBUF and (n_chunks - NBUF) % NBUF == 0
  mesh = plsc.VectorSubcoreMesh(core_axis_name="c", subcore_axis_name="s")

  @functools.partial(
      pl.kernel,
      out_type=jax.ShapeDtypeStruct((total_rows, dim), jnp.float32),
      mesh=mesh,
      scratch_types=[
          pltpu.VMEM((per_w,), jnp.int32),
          [pltpu.VMEM((CHUNK, dim), jnp.float32) for _ in range(NBUF)],
          [pltpu.SemaphoreType.DMA for _ in range(NBUF)],
          [pltpu.SemaphoreType.DMA for _ in range(NBUF)],
      ],
      compiler_params=pltpu.CompilerParams(use_tc_tiling_on_sc=False),
  )
  def gather_kernel(idx_hbm, table_hbm, out_hbm, idx_v, rows, gsem, ssem):
    wid = lax.axis_index("s") * NUM_CORES + lax.axis_index("c")
    base = wid * per_w

    pltpu.sync_copy(idx_hbm.at[pl.ds(base, per_w)], idx_v)

    def fire_gather(c, b):
      pltpu.async_copy(
          table_hbm.at[idx_v.at[pl.ds(c * CHUNK, CHUNK)]], rows[b], gsem[b])

    def fire_store(c, b):
      pltpu.async_copy(rows[b], out_hbm.at[pl.ds(base + c * CHUNK, CHUNK)],
                       ssem[b])

    def wait_gather(b):
      pltpu.make_async_copy(
          table_hbm.at[pl.ds(0, CHUNK)], rows[b], gsem[b]).wait()

    def wait_store(b):
      pltpu.make_async_copy(
          rows[b], out_hbm.at[pl.ds(0, CHUNK)], ssem[b]).wait()

    for b in range(NBUF):
      fire_gather(b, b)

    @pl.loop(0, n_chunks - NBUF, step=NBUF)
    def _(g0):
      for b in range(NBUF):
        wait_gather(b)
        fire_store(g0 + b, b)
      for b in range(NBUF):
        wait_store(b)
        fire_gather(g0 + b + NBUF, b)

    for b in range(NBUF):
      c = n_chunks - NBUF + b
      wait_gather(b)
      fire_store(c, b)
    for b in range(NBUF):
      wait_store(b)

  return gather_kernel


def kernel(input_ids, table):
  b, s = input_ids.shape
  d = table.shape[1]
  flat = input_ids.T.reshape(b * s).astype(jnp.int32)
  out = _make_gather(b * s, d)(flat, table)
  return out.reshape(s, b, d).transpose(1, 0, 2)
